# baseline (speedup 1.0000x reference)
import functools

import jax
import jax.numpy as jnp
from jax import lax
from jax.experimental import pallas as pl
from jax.experimental.pallas import tpu as pltpu
from jax.experimental.pallas import tpu_sc as plsc

N = 10000
N_IN = 128
N_H = 64
N_IDX = 5000

_NC = 2
_NS = 16
_NW = _NC * _NS
_LANES = 16
_CHUNK = 320
_HALF = 16 * _CHUNK
_TOT = 2 * _HALF
_TE_LAST = N_IDX - 15 * _CHUNK


def _norm_body(x_ref, w1_ref, b1_ref, w2_ref, b2_ref, out_ref):
    x = x_ref[...]
    h = jnp.dot(x, w1_ref[...], preferred_element_type=jnp.float32) + b1_ref[...]
    h = jnp.maximum(h, 0.0)
    xr = jnp.dot(h, w2_ref[...], preferred_element_type=jnp.float32) + b2_ref[...]
    d = x - xr
    ones = jnp.ones((1, N_IN), dtype=jnp.float32)
    s = jax.lax.dot_general(ones, d * d, (((1,), (1,)), ((), ())),
                            preferred_element_type=jnp.float32)
    out_ref[...] = jnp.sqrt(s.reshape(N))


def _row_norms(x, w1, b1, w2, b2):
    return pl.pallas_call(
        _norm_body,
        out_shape=jax.ShapeDtypeStruct((N,), jnp.float32),
    )(x, w1, b1.reshape(1, N_H), w2, b2.reshape(1, N_IN))


def _sc_body(nrm_hbm, idx_hbm, te_out, part_out, idx_v, val_v, acc_v, sem):
    wid = lax.axis_index("s") * _NC + lax.axis_index("c")
    base = wid * _CHUNK
    pltpu.sync_copy(idx_hbm.at[pl.ds(base, _CHUNK)], idx_v)
    pltpu.async_copy(nrm_hbm.at[idx_v], val_v, sem).wait()

    @pl.when(wid < 15)
    def _te_full():
        pltpu.sync_copy(val_v, te_out.at[pl.ds(base, _CHUNK)])

    @pl.when(wid == 15)
    def _te_tail():
        pltpu.sync_copy(val_v.at[pl.ds(0, _TE_LAST)],
                        te_out.at[pl.ds(15 * _CHUNK, _TE_LAST)])

    @pl.when(wid >= 16)
    def _tr():
        lanes = lax.iota(jnp.int32, _LANES)
        g0 = (wid - 16) * _CHUNK

        def body(j, acc):
            g = lanes + (g0 + j * _LANES)
            v = val_v[pl.ds(j * _LANES, _LANES)]
            return acc + jnp.where(g < N_IDX, v, 0.0)

        acc_v[...] = lax.fori_loop(0, _CHUNK // _LANES,
                                   body, jnp.zeros((_LANES,), jnp.float32))
        pltpu.sync_copy(acc_v, part_out.at[wid - 16])


def _sc_gather(nrm, idx_all):
    mesh = plsc.VectorSubcoreMesh(core_axis_name="c", subcore_axis_name="s")
    run = functools.partial(
        pl.kernel,
        mesh=mesh,
        out_type=[
            jax.ShapeDtypeStruct((N_IDX,), jnp.float32),
            jax.ShapeDtypeStruct((16, _LANES), jnp.float32),
        ],
        scratch_types=[
            pltpu.VMEM((_CHUNK,), jnp.int32),
            pltpu.VMEM((_CHUNK,), jnp.float32),
            pltpu.VMEM((_LANES,), jnp.float32),
            pltpu.SemaphoreType.DMA,
        ],
    )(_sc_body)
    return run(nrm, idx_all)


def kernel(seq1, adj, idx_train, idx_test, W_stru, b_stru,
           W_attr1, b_attr1, W_attr2, b_attr2):
    del adj, W_stru, b_stru
    nrm = _row_norms(seq1, W_attr1, b_attr1, W_attr2, b_attr2).reshape(N)
    pad = jnp.zeros((_HALF - N_IDX,), jnp.int32)
    idx_all = jnp.concatenate([idx_test.astype(jnp.int32), pad,
                               idx_train.astype(jnp.int32), pad])
    te, parts = _sc_gather(nrm, idx_all)
    loss = jnp.sum(parts) * (1.0 / N_IDX)
    return (loss, te)
